# Initial kernel scaffold; baseline (speedup 1.0000x reference)
import jax
import jax.numpy as jnp
from jax.experimental import pallas as pl


def kernel(x, edge_index, edge_attr, W, b):
    raise NotImplementedError("write your pallas kernel here")


# speedup vs baseline: 2.5207x; 2.5207x over previous
import functools

import jax
import jax.numpy as jnp
from jax import lax
from jax.experimental import pallas as pl
from jax.experimental.pallas import tpu as pltpu
from jax.experimental.pallas import tpu_sc as plsc

F32 = jnp.float32
NC = 2
NS = 16
PCH = 80


def _sc_sum_x(x, src_r, dst_r, n_out):
    N, D = x.shape
    NW, rpt, CH = src_r.shape
    NP = N + 8
    n_chunks = N // PCH
    chunk_rounds = -(-n_chunks // NS)

    mesh = plsc.VectorSubcoreMesh(core_axis_name="c", subcore_axis_name="s")

    @functools.partial(
        pl.kernel,
        mesh=mesh,
        out_type=jax.ShapeDtypeStruct((NC, n_out, D), F32),
        scratch_types=[
            pltpu.VMEM_SHARED((NP, D), F32),
            pltpu.VMEM((8, CH), jnp.int32),
            pltpu.VMEM((8, CH), jnp.int32),
            pltpu.VMEM((CH, D), F32),
            pltpu.SemaphoreType.DMA,
        ],
    )
    def kern(x_hbm, src_hbm, dst_hbm, sx_hbm, acc_x, src_b, dst_b, rowb, sem):
        c = lax.axis_index("c")
        s = lax.axis_index("s")
        wid = c * NS + s

        def _zero_rowb(i, _):
            for k in range(D // 16):
                rowb[i, pl.ds(k * 16, 16)] = jnp.zeros((16,), F32)
            return _
        lax.fori_loop(0, CH, _zero_rowb, None)

        for t in range(chunk_rounds):
            cid = s + t * NS

            @pl.when(cid < n_chunks)
            def _():
                pltpu.sync_copy(rowb.at[pl.ds(0, PCH)],
                                acc_x.at[pl.ds(cid * PCH, PCH)])

        @pl.when(s == 0)
        def _():
            pltpu.sync_copy(rowb.at[pl.ds(0, 8)], acc_x.at[pl.ds(N, 8)])
        plsc.subcore_barrier()

        def _blk_body(bi, _):
            pltpu.sync_copy(src_hbm.at[wid].at[pl.ds(bi * 8, 8)], src_b)
            pltpu.sync_copy(dst_hbm.at[wid].at[pl.ds(bi * 8, 8)], dst_b)
            for r in range(8):
                pltpu.async_copy(x_hbm.at[src_b.at[r]], rowb, sem).wait()
                pltpu.sync_copy(rowb, acc_x.at[dst_b.at[r]], add=True)
            return _
        lax.fori_loop(0, rpt // 8, _blk_body, None)

        plsc.subcore_barrier()

        for t in range(chunk_rounds):
            cid = s + t * NS

            @pl.when(cid < n_chunks)
            def _():
                rr = cid * PCH
                pltpu.sync_copy(acc_x.at[pl.ds(rr, PCH)], rowb.at[pl.ds(0, PCH)])
                pltpu.sync_copy(rowb.at[pl.ds(0, PCH)],
                                sx_hbm.at[c].at[pl.ds(rr, PCH)])

    return kern(x, src_r, dst_r)


def _sc_sum_e(dst_w, ea_t, n, de, n_out):
    N, DE = n, de
    NSW, rows, CH = dst_w.shape
    FH = DE // NC
    NP = n_out

    mesh = plsc.VectorSubcoreMesh(core_axis_name="c", subcore_axis_name="s")

    @functools.partial(
        pl.kernel,
        mesh=mesh,
        compiler_params=pltpu.CompilerParams(needs_layout_passes=False),
        out_type=(
            jax.ShapeDtypeStruct((NS * NC * FH * NP,), F32),
            jax.ShapeDtypeStruct((NS * NP,), F32),
        ),
        scratch_types=[
            pltpu.VMEM((8, CH), jnp.int32),
            pltpu.VMEM((FH, CH), F32),
            pltpu.VMEM((FH * NP,), F32),
            pltpu.VMEM((NP,), F32),
            pltpu.SemaphoreType.DMA,
        ],
    )
    def kern(dst_hbm, ea_hbm, se_hbm, dg_hbm, dst_b, ea_b, acc, accd, sem):
        c = lax.axis_index("c")
        s = lax.axis_index("s")

        def _zero(i, _):
            acc[pl.ds(i * 16, 16)] = jnp.zeros((16,), F32)
            return _
        lax.fori_loop(0, FH * NP // 16, _zero, None)

        def _zerod(i, _):
            accd[pl.ds(i * 16, 16)] = jnp.zeros((16,), F32)
            return _
        lax.fori_loop(0, NP // 16, _zerod, None)

        def _blk_body(bi, _):
            ones16 = jnp.ones((16,), F32)
            pltpu.sync_copy(dst_hbm.at[s].at[pl.ds(bi * 8, 8)], dst_b)
            for r in range(8):
                e0 = (s * rows + bi * 8 + r) * CH
                pltpu.sync_copy(
                    ea_hbm.at[pl.ds(c * FH, FH), pl.ds(e0, CH)], ea_b)
                for g in range(CH // 16):
                    dv = dst_b[r, pl.ds(g * 16, 16)]
                    for f in range(FH):
                        ev = ea_b[f, pl.ds(g * 16, 16)]
                        plsc.addupdate_scatter(acc, [dv + jnp.int32(f * NP)], ev)
                    plsc.addupdate_scatter(accd, [dv], ones16)
            return _
        lax.fori_loop(0, rows // 8, _blk_body, None)

        pltpu.sync_copy(acc, se_hbm.at[pl.ds((s * NC + c) * FH * NP, FH * NP)])

        @pl.when(c == 0)
        def _():
            pltpu.sync_copy(accd, dg_hbm.at[pl.ds(s * NP, NP)])

    return kern(dst_w, ea_t)


def _tc_finish(x, sx, se_t, dg_t, W, b2):
    D = x.shape[1]
    DE = se_t.shape[0] // NS

    NPAD = x.shape[0]
    BLK = 1280

    def body(x_ref, sx_ref, se_ref, dg_ref, w_ref, b_ref, o_ref):
        y = x_ref[...] + sx_ref[0] + sx_ref[1]
        zt = jnp.sum(se_ref[...].reshape(NS, DE, BLK), axis=0)
        d = 1.0 + jnp.sum(dg_ref[...], axis=0)[:, None]
        h = (jnp.dot(y, w_ref[0:D, :], preferred_element_type=F32)
             + lax.dot_general(zt, w_ref[D:, :], (((0,), (0,)), ((), ())),
                               preferred_element_type=F32)) / d
        h = h + b_ref[...]
        ss = jnp.sum(h * h, axis=-1, keepdims=True)
        o_ref[...] = h * lax.rsqrt(jnp.maximum(ss, 1e-12))

    return pl.pallas_call(
        body,
        grid=(NPAD // BLK,),
        in_specs=[
            pl.BlockSpec((BLK, D), lambda i: (i, 0)),
            pl.BlockSpec((NC, BLK, D), lambda i: (0, i, 0)),
            pl.BlockSpec((NS * DE, BLK), lambda i: (0, i)),
            pl.BlockSpec((NS, BLK), lambda i: (0, i)),
            pl.BlockSpec((D + DE, D), lambda i: (0, 0)),
            pl.BlockSpec((1, D), lambda i: (0, 0)),
        ],
        out_specs=pl.BlockSpec((BLK, D), lambda i: (i, 0)),
        out_shape=jax.ShapeDtypeStruct((NPAD, D), F32),
    )(x, sx, se_t, dg_t, W, b2)


def kernel(x, edge_index, edge_attr, W, b):
    N, D = x.shape
    E = edge_index.shape[1]
    DE = edge_attr.shape[1]
    CH = 128
    NW = NC * NS
    rpt = -(-E // (CH * NW))
    rpt += (-rpt) % 8
    e_pad = NW * rpt * CH
    pad = e_pad - E
    src_p = jnp.concatenate([edge_index[0], jnp.zeros((pad,), jnp.int32)])
    dst_p = jnp.concatenate([edge_index[1], jnp.full((pad,), N, jnp.int32)])
    src_r = src_p.reshape(NW, rpt, CH)
    dst_r = dst_p.reshape(NW, rpt, CH)
    dst_w = dst_p.reshape(NS, NW * rpt // NS, CH)
    ea_t = jnp.pad(edge_attr.T, ((0, 0), (0, pad)))
    NPAD = 1280 * (-(-N // 1280))
    sx = _sc_sum_x(x, src_r, dst_r, NPAD)
    dst_w, ea_t, sx = lax.optimization_barrier((dst_w, ea_t, sx))
    se_p, dg_p = _sc_sum_e(dst_w, ea_t, N, DE, NPAD)
    sx, se_p, dg_p = lax.optimization_barrier((sx, se_p, dg_p))
    se_t = se_p.reshape(NS * DE, NPAD)
    dg_t = dg_p.reshape(NS, NPAD)
    x_pad = jnp.pad(x, ((0, NPAD - N), (0, 0)))
    out = _tc_finish(x_pad, sx, se_t, dg_t, W, b.reshape(1, D))
    return out[:N]


# speedup vs baseline: 2.9056x; 1.1527x over previous
import functools

import jax
import jax.numpy as jnp
from jax import lax
from jax.experimental import pallas as pl
from jax.experimental.pallas import tpu as pltpu
from jax.experimental.pallas import tpu_sc as plsc

F32 = jnp.float32
NC = 2
NS = 16
PCH = 80


def _sc_sum_x(x, src_r, dst_r, n_out):
    N, D = x.shape
    NW, rpt, CH = src_r.shape
    NP = N + 8
    n_chunks = N // PCH
    chunk_rounds = -(-n_chunks // NS)

    mesh = plsc.VectorSubcoreMesh(core_axis_name="c", subcore_axis_name="s")

    @functools.partial(
        pl.kernel,
        mesh=mesh,
        out_type=jax.ShapeDtypeStruct((NC, n_out, D), F32),
        scratch_types=[
            pltpu.VMEM_SHARED((NP, D), F32),
            pltpu.VMEM((8, CH), jnp.int32),
            pltpu.VMEM((8, 2, CH // 2), jnp.int32),
            pltpu.VMEM((CH // 2, D), F32),
            pltpu.VMEM((CH // 2, D), F32),
            pltpu.SemaphoreType.DMA,
            pltpu.SemaphoreType.DMA,
        ],
    )
    def kern(x_hbm, src_hbm, dst_hbm, sx_hbm,
             acc_x, src_b, dst3, bufa, bufb, sema, semb):
        c = lax.axis_index("c")
        s = lax.axis_index("s")
        wid = c * NS + s
        H = CH // 2
        bufs = (bufa, bufb)
        sems = (sema, semb)

        def _zero_bufs(i, _):
            for k in range(D // 16):
                bufa[i, pl.ds(k * 16, 16)] = jnp.zeros((16,), F32)
            return _
        lax.fori_loop(0, H, _zero_bufs, None)

        for t in range(chunk_rounds):
            cid = s + t * NS

            @pl.when(cid < n_chunks)
            def _():
                rr = cid * PCH
                pltpu.sync_copy(bufa.at[pl.ds(0, H)], acc_x.at[pl.ds(rr, H)])
                pltpu.sync_copy(bufa.at[pl.ds(0, PCH - H)],
                                acc_x.at[pl.ds(rr + H, PCH - H)])

        @pl.when(s == 0)
        def _():
            pltpu.sync_copy(bufa.at[pl.ds(0, 8)], acc_x.at[pl.ds(N, 8)])
        plsc.subcore_barrier()

        def _blk_body(bi, _):
            pltpu.sync_copy(src_hbm.at[wid].at[pl.ds(bi * 8, 8)], src_b)
            pltpu.sync_copy(dst_hbm.at[wid].at[pl.ds(bi * 8, 8)], dst3)
            cps = [None, None]
            cps[0] = pltpu.async_copy(
                x_hbm.at[src_b.at[0, pl.ds(0, H)]], bufa, sema)
            for k in range(16):
                r, h = divmod(k, 2)
                if k < 15:
                    rn, hn = divmod(k + 1, 2)
                    cps[(k + 1) % 2] = pltpu.async_copy(
                        x_hbm.at[src_b.at[rn, pl.ds(hn * H, H)]],
                        bufs[(k + 1) % 2], sems[(k + 1) % 2])
                cps[k % 2].wait()
                pltpu.sync_copy(bufs[k % 2], acc_x.at[dst3.at[r, h]], add=True)
            return _
        lax.fori_loop(0, rpt // 8, _blk_body, None)

        plsc.subcore_barrier()

        for t in range(chunk_rounds):
            cid = s + t * NS

            @pl.when(cid < n_chunks)
            def _():
                rr = cid * PCH
                pltpu.sync_copy(acc_x.at[pl.ds(rr, H)], bufa.at[pl.ds(0, H)])
                pltpu.sync_copy(bufa.at[pl.ds(0, H)],
                                sx_hbm.at[c].at[pl.ds(rr, H)])
                pltpu.sync_copy(acc_x.at[pl.ds(rr + H, PCH - H)],
                                bufb.at[pl.ds(0, PCH - H)])
                pltpu.sync_copy(bufb.at[pl.ds(0, PCH - H)],
                                sx_hbm.at[c].at[pl.ds(rr + H, PCH - H)])

    return kern(x, src_r, dst_r)


def _sc_sum_e(dst_w, ea_t, n, de, n_out):
    N, DE = n, de
    NSW, rows, CH = dst_w.shape
    FH = DE // NC
    NP = n_out

    mesh = plsc.VectorSubcoreMesh(core_axis_name="c", subcore_axis_name="s")

    @functools.partial(
        pl.kernel,
        mesh=mesh,
        compiler_params=pltpu.CompilerParams(needs_layout_passes=False),
        out_type=(
            jax.ShapeDtypeStruct((NS * NC * FH * NP,), F32),
            jax.ShapeDtypeStruct((NS * NP,), F32),
        ),
        scratch_types=[
            pltpu.VMEM((8, CH), jnp.int32),
            pltpu.VMEM((FH, 8 * CH), F32),
            pltpu.VMEM((FH * NP,), F32),
            pltpu.VMEM((NP,), F32),
            pltpu.SemaphoreType.DMA,
        ],
    )
    def kern(dst_hbm, ea_hbm, se_hbm, dg_hbm, dst_b, ea_b, acc, accd, sem):
        c = lax.axis_index("c")
        s = lax.axis_index("s")

        def _zero(i, _):
            acc[pl.ds(i * 16, 16)] = jnp.zeros((16,), F32)
            return _
        lax.fori_loop(0, FH * NP // 16, _zero, None)

        def _zerod(i, _):
            accd[pl.ds(i * 16, 16)] = jnp.zeros((16,), F32)
            return _
        lax.fori_loop(0, NP // 16, _zerod, None)

        def _blk_body(bi, _):
            ones16 = jnp.ones((16,), F32)
            pltpu.sync_copy(dst_hbm.at[s].at[pl.ds(bi * 8, 8)], dst_b)
            e0 = (s * rows + bi * 8) * CH
            pltpu.sync_copy(
                ea_hbm.at[pl.ds(c * FH, FH), pl.ds(e0, 8 * CH)], ea_b)
            for r in range(8):
                for g in range(CH // 16):
                    dv = dst_b[r, pl.ds(g * 16, 16)]
                    for f in range(FH):
                        ev = ea_b[f, pl.ds(r * CH + g * 16, 16)]
                        plsc.addupdate_scatter(acc, [dv + jnp.int32(f * NP)], ev)
                    plsc.addupdate_scatter(accd, [dv], ones16)
            return _
        lax.fori_loop(0, rows // 8, _blk_body, None)

        pltpu.sync_copy(acc, se_hbm.at[pl.ds((s * NC + c) * FH * NP, FH * NP)])

        @pl.when(c == 0)
        def _():
            pltpu.sync_copy(accd, dg_hbm.at[pl.ds(s * NP, NP)])

    return kern(dst_w, ea_t)


def _tc_finish(x, sx, se_t, dg_t, W, b2):
    D = x.shape[1]
    DE = se_t.shape[0] // NS

    NPAD = x.shape[0]
    BLK = 1280

    def body(x_ref, sx_ref, se_ref, dg_ref, w_ref, b_ref, o_ref):
        y = x_ref[...] + sx_ref[0] + sx_ref[1]
        zt = jnp.sum(se_ref[...].reshape(NS, DE, BLK), axis=0)
        d = 1.0 + jnp.sum(dg_ref[...], axis=0)[:, None]
        h = (jnp.dot(y, w_ref[0:D, :], preferred_element_type=F32)
             + lax.dot_general(zt, w_ref[D:, :], (((0,), (0,)), ((), ())),
                               preferred_element_type=F32)) / d
        h = h + b_ref[...]
        ss = jnp.sum(h * h, axis=-1, keepdims=True)
        o_ref[...] = h * lax.rsqrt(jnp.maximum(ss, 1e-12))

    return pl.pallas_call(
        body,
        grid=(NPAD // BLK,),
        in_specs=[
            pl.BlockSpec((BLK, D), lambda i: (i, 0)),
            pl.BlockSpec((NC, BLK, D), lambda i: (0, i, 0)),
            pl.BlockSpec((NS * DE, BLK), lambda i: (0, i)),
            pl.BlockSpec((NS, BLK), lambda i: (0, i)),
            pl.BlockSpec((D + DE, D), lambda i: (0, 0)),
            pl.BlockSpec((1, D), lambda i: (0, 0)),
        ],
        out_specs=pl.BlockSpec((BLK, D), lambda i: (i, 0)),
        out_shape=jax.ShapeDtypeStruct((NPAD, D), F32),
    )(x, sx, se_t, dg_t, W, b2)


def kernel(x, edge_index, edge_attr, W, b):
    N, D = x.shape
    E = edge_index.shape[1]
    DE = edge_attr.shape[1]
    CH = 128
    NW = NC * NS
    rpt = -(-E // (CH * NW))
    rpt += (-rpt) % 8
    e_pad = NW * rpt * CH
    pad = e_pad - E
    src_p = jnp.concatenate([edge_index[0], jnp.zeros((pad,), jnp.int32)])
    dst_p = jnp.concatenate([edge_index[1], jnp.full((pad,), N, jnp.int32)])
    src_r = src_p.reshape(NW, rpt, CH)
    dst_r = dst_p.reshape(NW, rpt, 2, CH // 2)
    dst_w = dst_p.reshape(NS, NW * rpt // NS, CH)
    ea_t = jnp.pad(edge_attr.T, ((0, 0), (0, pad)))
    NPAD = 1280 * (-(-N // 1280))
    sx = _sc_sum_x(x, src_r, dst_r, NPAD)
    dst_w, ea_t, sx = lax.optimization_barrier((dst_w, ea_t, sx))
    se_p, dg_p = _sc_sum_e(dst_w, ea_t, N, DE, NPAD)
    sx, se_p, dg_p = lax.optimization_barrier((sx, se_p, dg_p))
    se_t = se_p.reshape(NS * DE, NPAD)
    dg_t = dg_p.reshape(NS, NPAD)
    x_pad = jnp.pad(x, ((0, NPAD - N), (0, 0)))
    out = _tc_finish(x_pad, sx, se_t, dg_t, W, b.reshape(1, D))
    return out[:N]


# speedup vs baseline: 3.0006x; 1.0327x over previous
import functools

import jax
import jax.numpy as jnp
from jax import lax
from jax.experimental import pallas as pl
from jax.experimental.pallas import tpu as pltpu
from jax.experimental.pallas import tpu_sc as plsc

F32 = jnp.float32
NC = 2
NS = 16
PCH = 80


def _sc_sum_x(x, src_r, dst_r, n_out):
    N, D = x.shape
    NW, rpt, CH = src_r.shape
    NP = N + 8
    n_chunks = N // PCH
    chunk_rounds = -(-n_chunks // NS)

    mesh = plsc.VectorSubcoreMesh(core_axis_name="c", subcore_axis_name="s")

    @functools.partial(
        pl.kernel,
        mesh=mesh,
        out_type=jax.ShapeDtypeStruct((NC, n_out, D), F32),
        scratch_types=[
            pltpu.VMEM_SHARED((NP, D), F32),
            pltpu.VMEM((8, CH), jnp.int32),
            pltpu.VMEM((8, 2, CH // 2), jnp.int32),
            pltpu.VMEM((CH // 2, D), F32),
            pltpu.VMEM((CH // 2, D), F32),
            pltpu.SemaphoreType.DMA,
            pltpu.SemaphoreType.DMA,
        ],
    )
    def kern(x_hbm, src_hbm, dst_hbm, sx_hbm,
             acc_x, src_b, dst3, bufa, bufb, sema, semb):
        c = lax.axis_index("c")
        s = lax.axis_index("s")
        wid = c * NS + s
        H = CH // 2
        bufs = (bufa, bufb)
        sems = (sema, semb)

        def _zero_bufs(i, _):
            for k in range(D // 16):
                bufa[i, pl.ds(k * 16, 16)] = jnp.zeros((16,), F32)
            return _
        lax.fori_loop(0, H, _zero_bufs, None)

        for t in range(chunk_rounds):
            cid = s + t * NS

            @pl.when(cid < n_chunks)
            def _():
                rr = cid * PCH
                pltpu.sync_copy(bufa.at[pl.ds(0, H)], acc_x.at[pl.ds(rr, H)])
                pltpu.sync_copy(bufa.at[pl.ds(0, PCH - H)],
                                acc_x.at[pl.ds(rr + H, PCH - H)])

        @pl.when(s == 0)
        def _():
            pltpu.sync_copy(bufa.at[pl.ds(0, 8)], acc_x.at[pl.ds(N, 8)])
        plsc.subcore_barrier()

        def _blk_body(bi, _):
            pltpu.sync_copy(src_hbm.at[wid].at[pl.ds(bi * 8, 8)], src_b)
            pltpu.sync_copy(dst_hbm.at[wid].at[pl.ds(bi * 8, 8)], dst3)
            cps = [None, None]
            cps[0] = pltpu.async_copy(
                x_hbm.at[src_b.at[0, pl.ds(0, H)]], bufa, sema)
            for k in range(16):
                r, h = divmod(k, 2)
                if k < 15:
                    rn, hn = divmod(k + 1, 2)
                    cps[(k + 1) % 2] = pltpu.async_copy(
                        x_hbm.at[src_b.at[rn, pl.ds(hn * H, H)]],
                        bufs[(k + 1) % 2], sems[(k + 1) % 2])
                cps[k % 2].wait()
                pltpu.sync_copy(bufs[k % 2], acc_x.at[dst3.at[r, h]], add=True)
            return _
        lax.fori_loop(0, rpt // 8, _blk_body, None)

        plsc.subcore_barrier()

        for t in range(chunk_rounds):
            cid = s + t * NS

            @pl.when(cid < n_chunks)
            def _():
                rr = cid * PCH
                pltpu.sync_copy(acc_x.at[pl.ds(rr, H)], bufa.at[pl.ds(0, H)])
                pltpu.sync_copy(bufa.at[pl.ds(0, H)],
                                sx_hbm.at[c].at[pl.ds(rr, H)])
                pltpu.sync_copy(acc_x.at[pl.ds(rr + H, PCH - H)],
                                bufb.at[pl.ds(0, PCH - H)])
                pltpu.sync_copy(bufb.at[pl.ds(0, PCH - H)],
                                sx_hbm.at[c].at[pl.ds(rr + H, PCH - H)])

    return kern(x, src_r, dst_r)


def _sc_sum_e(dst_w, ea_t, n, de, n_out):
    N, DE = n, de
    NSW, rows, CH = dst_w.shape
    FH = DE // NC
    NP = n_out

    mesh = plsc.VectorSubcoreMesh(core_axis_name="c", subcore_axis_name="s")

    @functools.partial(
        pl.kernel,
        mesh=mesh,
        compiler_params=pltpu.CompilerParams(needs_layout_passes=False),
        out_type=(
            jax.ShapeDtypeStruct((NS * NC * FH * NP,), F32),
            jax.ShapeDtypeStruct((NS * NP,), F32),
        ),
        scratch_types=[
            pltpu.VMEM((8, CH), jnp.int32),
            pltpu.VMEM((8, CH), jnp.int32),
            pltpu.VMEM((FH, 8 * CH), F32),
            pltpu.VMEM((FH, 8 * CH), F32),
            pltpu.VMEM((FH * NP,), F32),
            pltpu.VMEM((NP,), F32),
            pltpu.SemaphoreType.DMA,
            pltpu.SemaphoreType.DMA,
            pltpu.SemaphoreType.DMA,
            pltpu.SemaphoreType.DMA,
        ],
    )
    def kern(dst_hbm, ea_hbm, se_hbm, dg_hbm, dst_b0, dst_b1, ea_b0, ea_b1,
             acc, accd, semd0, semd1, seme0, seme1):
        c = lax.axis_index("c")
        s = lax.axis_index("s")
        dst_bufs = (dst_b0, dst_b1)
        ea_bufs = (ea_b0, ea_b1)
        semds = (semd0, semd1)
        semes = (seme0, seme1)

        def _zero(i, _):
            acc[pl.ds(i * 16, 16)] = jnp.zeros((16,), F32)
            return _
        lax.fori_loop(0, FH * NP // 16, _zero, None)

        def _zerod(i, _):
            accd[pl.ds(i * 16, 16)] = jnp.zeros((16,), F32)
            return _
        lax.fori_loop(0, NP // 16, _zerod, None)

        nblk = rows // 8

        def _fire(bi, slot):
            pltpu.async_copy(
                dst_hbm.at[s].at[pl.ds(bi * 8, 8)], dst_bufs[slot], semds[slot])
            pltpu.async_copy(
                ea_hbm.at[pl.ds(c * FH, FH),
                          pl.ds((s * rows + bi * 8) * CH, 8 * CH)],
                ea_bufs[slot], semes[slot])

        def _wait(bi, slot):
            pltpu.make_async_copy(
                dst_hbm.at[s].at[pl.ds(bi * 8, 8)], dst_bufs[slot],
                semds[slot]).wait()
            pltpu.make_async_copy(
                ea_hbm.at[pl.ds(c * FH, FH),
                          pl.ds((s * rows + bi * 8) * CH, 8 * CH)],
                ea_bufs[slot], semes[slot]).wait()

        def _process(slot):
            ones16 = jnp.ones((16,), F32)
            dst_b = dst_bufs[slot]
            ea_b = ea_bufs[slot]
            for r in range(8):
                for g in range(CH // 16):
                    dv = dst_b[r, pl.ds(g * 16, 16)]
                    for f in range(FH):
                        ev = ea_b[f, pl.ds(r * CH + g * 16, 16)]
                        plsc.addupdate_scatter(acc, [dv + jnp.int32(f * NP)], ev)
                    plsc.addupdate_scatter(accd, [dv], ones16)

        _fire(0, 0)

        def _super_body(i, _):
            b0 = 2 * i
            _fire(b0 + 1, 1)
            _wait(b0, 0)
            _process(0)

            @pl.when(b0 + 2 < nblk)
            def _():
                _fire(b0 + 2, 0)
            _wait(b0 + 1, 1)
            _process(1)
            return _
        lax.fori_loop(0, nblk // 2, _super_body, None)

        pltpu.sync_copy(acc, se_hbm.at[pl.ds((s * NC + c) * FH * NP, FH * NP)])

        @pl.when(c == 0)
        def _():
            pltpu.sync_copy(accd, dg_hbm.at[pl.ds(s * NP, NP)])

    return kern(dst_w, ea_t)


def _tc_finish(x, sx, se_t, dg_t, W, b2):
    D = x.shape[1]
    DE = se_t.shape[0] // NS

    NPAD = x.shape[0]
    BLK = 1280

    def body(x_ref, sx_ref, se_ref, dg_ref, w_ref, b_ref, o_ref):
        y = x_ref[...] + sx_ref[0] + sx_ref[1]
        zt = jnp.sum(se_ref[...].reshape(NS, DE, BLK), axis=0)
        d = 1.0 + jnp.sum(dg_ref[...], axis=0)[:, None]
        h = (jnp.dot(y, w_ref[0:D, :], preferred_element_type=F32)
             + lax.dot_general(zt, w_ref[D:, :], (((0,), (0,)), ((), ())),
                               preferred_element_type=F32)) / d
        h = h + b_ref[...]
        ss = jnp.sum(h * h, axis=-1, keepdims=True)
        o_ref[...] = h * lax.rsqrt(jnp.maximum(ss, 1e-12))

    return pl.pallas_call(
        body,
        grid=(NPAD // BLK,),
        in_specs=[
            pl.BlockSpec((BLK, D), lambda i: (i, 0)),
            pl.BlockSpec((NC, BLK, D), lambda i: (0, i, 0)),
            pl.BlockSpec((NS * DE, BLK), lambda i: (0, i)),
            pl.BlockSpec((NS, BLK), lambda i: (0, i)),
            pl.BlockSpec((D + DE, D), lambda i: (0, 0)),
            pl.BlockSpec((1, D), lambda i: (0, 0)),
        ],
        out_specs=pl.BlockSpec((BLK, D), lambda i: (i, 0)),
        out_shape=jax.ShapeDtypeStruct((NPAD, D), F32),
    )(x, sx, se_t, dg_t, W, b2)


def kernel(x, edge_index, edge_attr, W, b):
    N, D = x.shape
    E = edge_index.shape[1]
    DE = edge_attr.shape[1]
    CH = 128
    NW = NC * NS
    rpt = -(-E // (CH * NW))
    rpt += (-rpt) % 8
    e_pad = NW * rpt * CH
    pad = e_pad - E
    src_p = jnp.concatenate([edge_index[0], jnp.zeros((pad,), jnp.int32)])
    dst_p = jnp.concatenate([edge_index[1], jnp.full((pad,), N, jnp.int32)])
    src_r = src_p.reshape(NW, rpt, CH)
    dst_r = dst_p.reshape(NW, rpt, 2, CH // 2)
    dst_w = dst_p.reshape(NS, NW * rpt // NS, CH)
    ea_t = jnp.pad(edge_attr.T, ((0, 0), (0, pad)))
    NPAD = 1280 * (-(-N // 1280))
    sx = _sc_sum_x(x, src_r, dst_r, NPAD)
    dst_w, ea_t, sx = lax.optimization_barrier((dst_w, ea_t, sx))
    se_p, dg_p = _sc_sum_e(dst_w, ea_t, N, DE, NPAD)
    sx, se_p, dg_p = lax.optimization_barrier((sx, se_p, dg_p))
    se_t = se_p.reshape(NS * DE, NPAD)
    dg_t = dg_p.reshape(NS, NPAD)
    x_pad = jnp.pad(x, ((0, NPAD - N), (0, 0)))
    out = _tc_finish(x_pad, sx, se_t, dg_t, W, b.reshape(1, D))
    return out[:N]
